# baseline (speedup 1.0000x reference)
import functools

import jax
import jax.numpy as jnp
from jax import lax
from jax.experimental import pallas as pl
from jax.experimental.pallas import tpu as pltpu
from jax.experimental.pallas import tpu_sc as plsc


@functools.lru_cache(maxsize=None)
def _make_gather(V, D, B):
    info = plsc.get_sparse_core_info()
    NC, NS = info.num_cores, info.num_subcores
    NW = NC * NS
    assert B % NW == 0
    b_per_w = B // NW
    C = 16
    NBUF = 4
    assert b_per_w % (C * NBUF) == 0
    n_chunks = b_per_w // C
    n_groups = n_chunks // NBUF

    mesh = plsc.VectorSubcoreMesh(core_axis_name="c", subcore_axis_name="s")

    @functools.partial(
        pl.kernel,
        mesh=mesh,
        out_type=jax.ShapeDtypeStruct((B, D), jnp.float32),
        scratch_types=[
            pltpu.VMEM((b_per_w,), jnp.int32),
            pltpu.VMEM((NBUF, C, D), jnp.float32),
        ]
        + [pltpu.SemaphoreType.DMA] * (2 * NBUF),
    )
    def gather_kernel(idx_hbm, table_hbm, out_hbm, idx_v, rows_v, *sems):
        sem_in, sem_out = sems[:NBUF], sems[NBUF:]
        wid = lax.axis_index("s") * NC + lax.axis_index("c")
        base = wid * b_per_w
        pltpu.sync_copy(idx_hbm.at[pl.ds(base, b_per_w)], idx_v)

        def in_desc(chunk, b):
            return pltpu.make_async_copy(
                table_hbm.at[idx_v.at[pl.ds(chunk * C, C)]], rows_v.at[b], sem_in[b]
            )

        def out_desc(chunk, b):
            return pltpu.make_async_copy(
                rows_v.at[b], out_hbm.at[pl.ds(base + chunk * C, C)], sem_out[b]
            )

        for b in range(NBUF):
            in_desc(b, b).start()

        def body(g, carry):
            for b in range(NBUF):
                chunk = g * NBUF + b
                in_desc(chunk, b).wait()
                out_desc(chunk, b).start()
            for b in range(NBUF):
                chunk = g * NBUF + b
                out_desc(chunk, b).wait()

                @pl.when(g < n_groups - 1)
                def _():
                    in_desc(chunk + NBUF, b).start()

            return carry

        lax.fori_loop(0, n_groups, body, 0)

    return gather_kernel


def kernel(tokens, W_E):
    B = tokens.shape[0] * tokens.shape[1]
    V, D = W_E.shape
    flat = tokens.reshape(B).astype(jnp.int32)
    out = _make_gather(V, D, B)(flat, W_E)
    return out.reshape(tokens.shape[0], tokens.shape[1], D)


# speedup vs baseline: 1.0485x; 1.0485x over previous
import functools

import jax
import jax.numpy as jnp
from jax import lax
from jax.experimental import pallas as pl
from jax.experimental.pallas import tpu as pltpu
from jax.experimental.pallas import tpu_sc as plsc


@functools.lru_cache(maxsize=None)
def _make_gather(V, D, B):
    info = plsc.get_sparse_core_info()
    NC, NS = info.num_cores, info.num_subcores
    NW = NC * NS
    assert B % NW == 0
    b_per_w = B // NW
    C = 16
    NBUF = 7
    assert b_per_w % C == 0
    n_chunks = b_per_w // C

    mesh = plsc.VectorSubcoreMesh(core_axis_name="c", subcore_axis_name="s")

    @functools.partial(
        pl.kernel,
        mesh=mesh,
        out_type=jax.ShapeDtypeStruct((B, D), jnp.float32),
        scratch_types=[
            pltpu.VMEM((b_per_w,), jnp.int32),
            pltpu.VMEM((NBUF, C, D), jnp.float32),
        ]
        + [pltpu.SemaphoreType.DMA] * (2 * NBUF),
    )
    def gather_kernel(idx_hbm, table_hbm, out_hbm, idx_v, rows_v, *sems):
        sem_in, sem_out = sems[:NBUF], sems[NBUF:]
        wid = lax.axis_index("s") * NC + lax.axis_index("c")
        base = wid * b_per_w
        pltpu.sync_copy(idx_hbm.at[pl.ds(base, b_per_w)], idx_v)

        def in_desc(chunk, b):
            return pltpu.make_async_copy(
                table_hbm.at[idx_v.at[pl.ds(chunk * C, C)]], rows_v.at[b], sem_in[b]
            )

        def out_desc(chunk, b):
            return pltpu.make_async_copy(
                rows_v.at[b], out_hbm.at[pl.ds(base + chunk * C, C)], sem_out[b]
            )

        P = NBUF - 1
        for g in range(min(P, n_chunks)):
            in_desc(g, g % NBUF).start()
        for g in range(n_chunks):
            b = g % NBUF
            in_desc(g, b).wait()
            out_desc(g, b).start()
            nxt = g + P
            if nxt < n_chunks:
                old = nxt - NBUF
                if old >= 0:
                    out_desc(old, old % NBUF).wait()
                in_desc(nxt, nxt % NBUF).start()
        for g in range(max(0, n_chunks - NBUF), n_chunks):
            out_desc(g, g % NBUF).wait()

    return gather_kernel


def kernel(tokens, W_E):
    B = tokens.shape[0] * tokens.shape[1]
    V, D = W_E.shape
    flat = tokens.reshape(B).astype(jnp.int32)
    out = _make_gather(V, D, B)(flat, W_E)
    return out.reshape(tokens.shape[0], tokens.shape[1], D)
